# Initial kernel scaffold; baseline (speedup 1.0000x reference)
import jax
import jax.numpy as jnp
from jax.experimental import pallas as pl


def kernel(data, embedding_weight):
    raise NotImplementedError("write your pallas kernel here")


# speedup vs baseline: 2.2348x; 2.2348x over previous
import jax
import jax.numpy as jnp
from jax.experimental import pallas as pl

_T = 50
_F = 48
_E = 16
_BLOCK_ROWS = 3200


def _concat_body(d_ref, e_ref, o_ref):
    o_ref[:, :_F] = d_ref[:, :]
    o_ref[:, _F:] = e_ref[:, :]


def kernel(data, embedding_weight):
    g, n, t, f = data.shape
    assert t == _T and f == _F and embedding_weight.shape == (_T, _E)
    rows = g * n * t
    block = _BLOCK_ROWS
    assert rows % block == 0 and block % _T == 0

    d2 = data.reshape(rows, f)
    e2 = jnp.tile(embedding_weight, (block // _T, 1))

    out = pl.pallas_call(
        _concat_body,
        grid=(rows // block,),
        in_specs=[
            pl.BlockSpec((block, _F), lambda i: (i, 0)),
            pl.BlockSpec((block, _E), lambda i: (0, 0)),
        ],
        out_specs=pl.BlockSpec((block, _F + _E), lambda i: (i, 0)),
        out_shape=jax.ShapeDtypeStruct((rows, _F + _E), data.dtype),
    )(d2, e2)
    return out.reshape(g, n, t, _F + _E)


# speedup vs baseline: 2.5959x; 1.1616x over previous
import jax
import jax.numpy as jnp
from jax.experimental import pallas as pl

_T = 50
_F = 48
_E = 16
_BLOCK_ROWS = 256


def _concat_body(d_ref, p_ref, o_ref):
    b = d_ref.shape[0]
    o_ref[:, :, 0:_F] = d_ref[:, :, 0:_F]
    o_ref[:, :, 64 : 64 + _F] = d_ref[:, :, _F : 2 * _F]
    pat = p_ref[:, :]
    o_ref[:, :, _F:64] = jnp.broadcast_to(pat[None, :, 0:_E], (b, _T // 2, _E))
    o_ref[:, :, 64 + _F : 128] = jnp.broadcast_to(pat[None, :, _E:], (b, _T // 2, _E))


def kernel(data, embedding_weight):
    g, n, t, f = data.shape
    assert t == _T and f == _F and embedding_weight.shape == (_T, _E)
    rows = g * n
    block = _BLOCK_ROWS
    assert rows % block == 0

    d3 = data.reshape(rows, t // 2, 2 * f)
    pat = embedding_weight.reshape(t // 2, 2 * _E)

    out = pl.pallas_call(
        _concat_body,
        grid=(rows // block,),
        in_specs=[
            pl.BlockSpec((block, t // 2, 2 * f), lambda i: (i, 0, 0)),
            pl.BlockSpec((t // 2, 2 * _E), lambda i: (0, 0)),
        ],
        out_specs=pl.BlockSpec((block, t // 2, 128), lambda i: (i, 0, 0)),
        out_shape=jax.ShapeDtypeStruct((rows, t // 2, 128), data.dtype),
    )(d3, pat)
    return out.reshape(g, n, t, f + _E)


# speedup vs baseline: 2.6492x; 1.0206x over previous
import jax
import jax.numpy as jnp
from jax.experimental import pallas as pl

_T = 50
_F = 48
_E = 16
_BLOCK_ROWS = 512


def _concat_body(d_ref, p_ref, o_ref):
    b = d_ref.shape[0]
    d = d_ref[:, :, :]
    pat = p_ref[:, :]
    ea = jnp.broadcast_to(pat[None, :, 0:_E], (b, _T // 2, _E))
    eb = jnp.broadcast_to(pat[None, :, _E:], (b, _T // 2, _E))
    o_ref[:, :, :] = jnp.concatenate(
        (d[:, :, 0:_F], ea, d[:, :, _F : 2 * _F], eb), axis=2
    )


def kernel(data, embedding_weight):
    g, n, t, f = data.shape
    assert t == _T and f == _F and embedding_weight.shape == (_T, _E)
    rows = g * n
    block = _BLOCK_ROWS
    assert rows % block == 0

    d3 = data.reshape(rows, t // 2, 2 * f)
    pat = embedding_weight.reshape(t // 2, 2 * _E)

    out = pl.pallas_call(
        _concat_body,
        grid=(rows // block,),
        in_specs=[
            pl.BlockSpec((block, t // 2, 2 * f), lambda i: (i, 0, 0)),
            pl.BlockSpec((t // 2, 2 * _E), lambda i: (0, 0)),
        ],
        out_specs=pl.BlockSpec((block, t // 2, 128), lambda i: (i, 0, 0)),
        out_shape=jax.ShapeDtypeStruct((rows, t // 2, 128), data.dtype),
    )(d3, pat)
    return out.reshape(g, n, t, f + _E)
